# baseline (speedup 1.0000x reference)
import functools

import jax
import jax.numpy as jnp
from jax import lax
from jax.experimental import pallas as pl
from jax.experimental.pallas import tpu as pltpu
from jax.experimental.pallas import tpu_sc as plsc

VOCAB = 1000000
DIM = 16
B = 16384
NEG = 5

NC = 2
NS = 16
NW = NC * NS
CH = 128


def _make_sc_gather(n):
    rpw = n // NW
    nch = rpw // CH
    mesh = plsc.VectorSubcoreMesh(core_axis_name="c", subcore_axis_name="s")

    @functools.partial(
        pl.kernel,
        mesh=mesh,
        compiler_params=pltpu.CompilerParams(use_tc_tiling_on_sc=False),
        out_type=jax.ShapeDtypeStruct((n, DIM), jnp.float32),
        scratch_types=[
            pltpu.VMEM((nch, CH), jnp.int32),
            pltpu.VMEM((rpw, DIM), jnp.float32),
            pltpu.SemaphoreType.DMA,
        ],
    )
    def k(tab_hbm, idx_hbm, out_hbm, idx_v, rows_v, sem):
        wid = lax.axis_index("s") * NC + lax.axis_index("c")
        pltpu.sync_copy(idx_hbm.at[pl.ds(wid * nch, nch)], idx_v)
        descs = []
        for j in range(nch):
            descs.append(pltpu.async_copy(
                tab_hbm.at[idx_v.at[j]], rows_v.at[pl.ds(j * CH, CH)], sem))
        for d in descs:
            d.wait()
        pltpu.sync_copy(rows_v, out_hbm.at[pl.ds(wid * rpw, rpw)])

    return k


def _tc_score(u_rows3, vx_rows3):
    NBLK = 16
    BB = B // NBLK
    BBS = BB // 8

    def body(u_ref, v0, n1, n2, n3, n4, n5, out_ref, a_pos, a_xs, a_m, a_e):
        i = pl.program_id(0)
        u = u_ref[...].reshape(BB, DIM)
        s = jnp.sum(u * v0[...].reshape(BB, DIM), axis=1)
        ls = jnp.minimum(s, 0.0) - jnp.log1p(jnp.exp(-jnp.abs(s)))
        negsum = (n1[...] + n2[...] + n3[...] + n4[...] + n5[...]
                  ).reshape(BB, DIM)
        x = -jnp.sum(negsum * u, axis=1)
        bmax = jnp.max(x)
        bpos = jnp.full((1, 128), jnp.sum(ls), jnp.float32)
        bxs = jnp.full((1, 128), jnp.sum(x), jnp.float32)
        bm = jnp.full((1, 128), bmax, jnp.float32)
        be = jnp.full((1, 128), jnp.sum(jnp.exp(x - bmax)), jnp.float32)

        @pl.when(i == 0)
        def _():
            a_pos[...] = bpos
            a_xs[...] = bxs
            a_m[...] = bm
            a_e[...] = be

        @pl.when(i > 0)
        def _():
            m_old = a_m[...]
            m_new = jnp.maximum(m_old, bm)
            a_e[...] = a_e[...] * jnp.exp(m_old - m_new) + be * jnp.exp(bm - m_new)
            a_m[...] = m_new
            a_pos[...] = a_pos[...] + bpos
            a_xs[...] = a_xs[...] + bxs

        @pl.when(i == NBLK - 1)
        def _():
            lse = a_m[...] + jnp.log(a_e[...])
            loss_neg = a_xs[...] - jnp.float32(B) * lse
            out_ref[...] = -(a_pos[...] + loss_neg)

    out = pl.pallas_call(
        body,
        grid=(NBLK,),
        in_specs=[pl.BlockSpec((BBS, 8, DIM), lambda i: (i, 0, 0))]
        + [pl.BlockSpec((BBS, 8, DIM), (lambda i, k=k: (k * NBLK + i, 0, 0)))
           for k in range(NEG + 1)],
        out_specs=pl.BlockSpec((1, 128), lambda i: (0, 0)),
        out_shape=jax.ShapeDtypeStruct((1, 128), jnp.float32),
        scratch_shapes=[pltpu.VMEM((1, 128), jnp.float32) for _ in range(4)],
    )(u_rows3, vx_rows3, vx_rows3, vx_rows3, vx_rows3, vx_rows3, vx_rows3)
    return out[0, 0]


def kernel(center, context, neg_v, u_emb, v_emb):
    center = center.astype(jnp.int32)
    context = context.astype(jnp.int32)
    neg_v = neg_v.astype(jnp.int32)
    vx_idx = jnp.concatenate([context, jnp.swapaxes(neg_v, 0, 1).reshape(-1)])
    cidx2 = center.reshape(B // CH, CH)
    vxidx2 = vx_idx.reshape((NEG + 1) * B // CH, CH)
    vx_rows = _make_sc_gather((NEG + 1) * B)(v_emb, vxidx2)
    u_rows = _make_sc_gather(B)(u_emb, cidx2)
    nv = (NEG + 1) * B
    return _tc_score(u_rows.reshape(B // 8, 8, DIM),
                     vx_rows.reshape(nv // 8, 8, DIM))
